# initial kernel scaffold (unmeasured)
import jax
import jax.numpy as jnp
from jax import lax
from jax.experimental import pallas as pl
from jax.experimental.pallas import tpu as pltpu


def kernel(
    x,
):
    def body(*refs):
        pass

    out_shape = jax.ShapeDtypeStruct(..., jnp.float32)
    return pl.pallas_call(body, out_shape=out_shape)(...)


# baseline (device time: 70870 ns/iter reference)
import jax
import jax.numpy as jnp
from jax import lax
from jax.experimental import pallas as pl
from jax.experimental.pallas import tpu as pltpu

N_DEV = 8


def kernel(x):
    m, n = x.shape
    c = m // N_DEV

    def body(x_ref, out_ref, comm_ref, rs_send_sems, rs_recv_sems,
             ag_send_sems, ag_recv_sems):
        my = lax.axis_index("i")
        left = lax.rem(my + N_DEV - 1, N_DEV)
        right = lax.rem(my + 1, N_DEV)

        barrier_sem = pltpu.get_barrier_semaphore()
        for nbr in (left, right):
            pl.semaphore_signal(
                barrier_sem, inc=1,
                device_id=(nbr,), device_id_type=pl.DeviceIdType.MESH,
            )
        pl.semaphore_wait(barrier_sem, 2)

        out_ref[...] = x_ref[...]

        for s in range(N_DEV - 1):
            send_chunk = lax.rem(my + N_DEV - s, N_DEV)
            recv_chunk = lax.rem(my + 2 * N_DEV - s - 1, N_DEV)
            rdma = pltpu.make_async_remote_copy(
                src_ref=out_ref.at[pl.ds(send_chunk * c, c), :],
                dst_ref=comm_ref.at[s],
                send_sem=rs_send_sems.at[s],
                recv_sem=rs_recv_sems.at[s],
                device_id=(right,),
                device_id_type=pl.DeviceIdType.MESH,
            )
            rdma.start()
            rdma.wait()
            out_ref[pl.ds(recv_chunk * c, c), :] += comm_ref[s]

        for s in range(N_DEV - 1):
            chunk = lax.rem(my + 2 * N_DEV + 1 - s, N_DEV)
            rdma = pltpu.make_async_remote_copy(
                src_ref=out_ref.at[pl.ds(chunk * c, c), :],
                dst_ref=out_ref.at[pl.ds(chunk * c, c), :],
                send_sem=ag_send_sems.at[s],
                recv_sem=ag_recv_sems.at[s],
                device_id=(right,),
                device_id_type=pl.DeviceIdType.MESH,
            )
            rdma.start()
            rdma.wait()

    return pl.pallas_call(
        body,
        out_shape=jax.ShapeDtypeStruct((m, n), x.dtype),
        in_specs=[pl.BlockSpec(memory_space=pltpu.VMEM)],
        out_specs=pl.BlockSpec(memory_space=pltpu.VMEM),
        scratch_shapes=[
            pltpu.VMEM((N_DEV - 1, c, n), x.dtype),
            pltpu.SemaphoreType.DMA((N_DEV - 1,)),
            pltpu.SemaphoreType.DMA((N_DEV - 1,)),
            pltpu.SemaphoreType.DMA((N_DEV - 1,)),
            pltpu.SemaphoreType.DMA((N_DEV - 1,)),
        ],
        compiler_params=pltpu.CompilerParams(collective_id=0),
    )(x)


# device time: 32098 ns/iter; 2.2079x vs baseline; 2.2079x over previous
import jax
import jax.numpy as jnp
from jax import lax
from jax.experimental import pallas as pl
from jax.experimental.pallas import tpu as pltpu

N_DEV = 8
M, N = 1024, 512

PARTS = (
    (0, 384, ("x", "y", "z")),
    (384, 320, ("y", "z", "x")),
    (704, 320, ("z", "x", "y")),
)


def kernel(x):
    m, n = x.shape
    assert (m, n) == (M, N)
    n_parts = len(PARTS)

    comm_offs = []
    off = 0
    for _, size, _ in PARTS:
        offs = []
        for frac in (2, 4, 8):
            offs.append(off)
            off += size // frac
        comm_offs.append(offs)
    comm_rows = off

    def body(x_ref, out_ref, comm_ref, send_sems, recv_sems):
        my = lax.axis_index("i")
        bz = my // 4
        q = lax.rem(my, 4)
        by = q // 2
        bx = jnp.bitwise_xor(by, lax.rem(q, 2))

        bits = {"x": bx, "y": by, "z": bz}
        partners = {
            "x": bz * 4 + by * 2 + jnp.bitwise_xor(1 - bx, by),
            "y": bz * 4 + (1 - by) * 2 + jnp.bitwise_xor(bx, 1 - by),
            "z": (1 - bz) * 4 + by * 2 + jnp.bitwise_xor(bx, by),
        }

        barrier_sem = pltpu.get_barrier_semaphore()
        for ax in ("x", "y", "z"):
            pl.semaphore_signal(
                barrier_sem, inc=1,
                device_id=(partners[ax],), device_id_type=pl.DeviceIdType.MESH,
            )
        pl.semaphore_wait(barrier_sem, 3)

        out_ref[...] = x_ref[...]

        keep_offs = [[None] * 3 for _ in range(n_parts)]
        send_offs = [[None] * 3 for _ in range(n_parts)]
        for p, (base, size, order) in enumerate(PARTS):
            cur = base
            for lvl, ax in enumerate(order):
                half = size >> (lvl + 1)
                keep_offs[p][lvl] = cur + bits[ax] * half
                send_offs[p][lvl] = cur + (1 - bits[ax]) * half
                cur = keep_offs[p][lvl]

        def sem_idx(stage, p):
            return stage * n_parts + p

        for lvl in range(3):
            rdmas = []
            for p, (base, size, order) in enumerate(PARTS):
                sz = size >> (lvl + 1)
                rdma = pltpu.make_async_remote_copy(
                    src_ref=out_ref.at[pl.ds(send_offs[p][lvl], sz), :],
                    dst_ref=comm_ref.at[pl.ds(comm_offs[p][lvl], sz), :],
                    send_sem=send_sems.at[sem_idx(lvl, p)],
                    recv_sem=recv_sems.at[sem_idx(lvl, p)],
                    device_id=(partners[order[lvl]],),
                    device_id_type=pl.DeviceIdType.MESH,
                )
                rdma.start()
                rdmas.append(rdma)
            for p, (base, size, order) in enumerate(PARTS):
                sz = size >> (lvl + 1)
                rdmas[p].wait()
                out_ref[pl.ds(keep_offs[p][lvl], sz), :] += comm_ref[
                    pl.ds(comm_offs[p][lvl], sz), :
                ]

        for stage, lvl in enumerate((2, 1, 0), start=3):
            rdmas = []
            for p, (base, size, order) in enumerate(PARTS):
                sz = size >> (lvl + 1)
                rdma = pltpu.make_async_remote_copy(
                    src_ref=out_ref.at[pl.ds(keep_offs[p][lvl], sz), :],
                    dst_ref=out_ref.at[pl.ds(keep_offs[p][lvl], sz), :],
                    send_sem=send_sems.at[sem_idx(stage, p)],
                    recv_sem=recv_sems.at[sem_idx(stage, p)],
                    device_id=(partners[order[lvl]],),
                    device_id_type=pl.DeviceIdType.MESH,
                )
                rdma.start()
                rdmas.append(rdma)
            for rdma in rdmas:
                rdma.wait()

    n_sems = 6 * n_parts
    return pl.pallas_call(
        body,
        out_shape=jax.ShapeDtypeStruct((m, n), x.dtype),
        in_specs=[pl.BlockSpec(memory_space=pltpu.VMEM)],
        out_specs=pl.BlockSpec(memory_space=pltpu.VMEM),
        scratch_shapes=[
            pltpu.VMEM((comm_rows, n), x.dtype),
            pltpu.SemaphoreType.DMA((n_sems,)),
            pltpu.SemaphoreType.DMA((n_sems,)),
        ],
        compiler_params=pltpu.CompilerParams(collective_id=0),
    )(x)


# device time: 30122 ns/iter; 2.3528x vs baseline; 1.0656x over previous
import jax
import jax.numpy as jnp
from jax import lax
from jax.experimental import pallas as pl
from jax.experimental.pallas import tpu as pltpu

N_DEV = 8
M, N = 1024, 512

PARTS = (
    (0, 384, ("x", "y", "z")),
    (384, 320, ("y", "z", "x")),
    (704, 320, ("z", "x", "y")),
)


def kernel(x):
    m, n = x.shape
    assert (m, n) == (M, N)
    n_parts = len(PARTS)

    comm_offs = []
    off = 0
    for _, size, _ in PARTS:
        offs = []
        for frac in (2, 4, 8):
            offs.append(off)
            off += size // frac
        comm_offs.append(offs)
    comm_rows = off

    def body(x_ref, out_ref, comm_ref, send_sems, recv_sems):
        my = lax.axis_index("i")
        bz = my // 4
        q = lax.rem(my, 4)
        by = q // 2
        bx = jnp.bitwise_xor(by, lax.rem(q, 2))

        bits = {"x": bx, "y": by, "z": bz}
        partners = {
            "x": bz * 4 + by * 2 + jnp.bitwise_xor(1 - bx, by),
            "y": bz * 4 + (1 - by) * 2 + jnp.bitwise_xor(bx, 1 - by),
            "z": (1 - bz) * 4 + by * 2 + jnp.bitwise_xor(bx, by),
        }

        barrier_sem = pltpu.get_barrier_semaphore()
        for ax in ("x", "y", "z"):
            pl.semaphore_signal(
                barrier_sem, inc=1,
                device_id=(partners[ax],), device_id_type=pl.DeviceIdType.MESH,
            )
        pl.semaphore_wait(barrier_sem, 3)

        keep_offs = [[None] * 3 for _ in range(n_parts)]
        send_offs = [[None] * 3 for _ in range(n_parts)]
        for p, (base, size, order) in enumerate(PARTS):
            cur = base
            for lvl, ax in enumerate(order):
                half = size >> (lvl + 1)
                keep_offs[p][lvl] = cur + bits[ax] * half
                send_offs[p][lvl] = cur + (1 - bits[ax]) * half
                cur = keep_offs[p][lvl]

        def sem_idx(stage, p):
            return stage * n_parts + p

        def start_rs(lvl, p):
            _, size, order = PARTS[p]
            sz = size >> (lvl + 1)
            src = x_ref if lvl == 0 else out_ref
            rdma = pltpu.make_async_remote_copy(
                src_ref=src.at[pl.ds(send_offs[p][lvl], sz), :],
                dst_ref=comm_ref.at[pl.ds(comm_offs[p][lvl], sz), :],
                send_sem=send_sems.at[sem_idx(lvl, p)],
                recv_sem=recv_sems.at[sem_idx(lvl, p)],
                device_id=(partners[order[lvl]],),
                device_id_type=pl.DeviceIdType.MESH,
            )
            rdma.start()
            return rdma

        def start_ag(lvl, p):
            _, size, order = PARTS[p]
            sz = size >> (lvl + 1)
            rdma = pltpu.make_async_remote_copy(
                src_ref=out_ref.at[pl.ds(keep_offs[p][lvl], sz), :],
                dst_ref=out_ref.at[pl.ds(keep_offs[p][lvl], sz), :],
                send_sem=send_sems.at[sem_idx(5 - lvl, p)],
                recv_sem=recv_sems.at[sem_idx(5 - lvl, p)],
                device_id=(partners[order[lvl]],),
                device_id_type=pl.DeviceIdType.MESH,
            )
            rdma.start()
            return rdma

        porder = (1, 2, 0)
        inflight = []

        rs = {p: start_rs(0, p) for p in porder}
        for lvl in range(3):
            nxt = {}
            for p in porder:
                _, size, _ = PARTS[p]
                sz = size >> (lvl + 1)
                rs[p].wait_recv()
                inflight.append(rs[p])
                if lvl == 0:
                    out_ref[pl.ds(keep_offs[p][0], sz), :] = (
                        x_ref[pl.ds(keep_offs[p][0], sz), :]
                        + comm_ref[pl.ds(comm_offs[p][0], sz), :]
                    )
                else:
                    out_ref[pl.ds(keep_offs[p][lvl], sz), :] += comm_ref[
                        pl.ds(comm_offs[p][lvl], sz), :
                    ]
                nxt[p] = start_rs(lvl + 1, p) if lvl < 2 else start_ag(2, p)
            rs = nxt

        ag = rs
        for lvl in (2, 1, 0):
            nxt = {}
            for p in porder:
                ag[p].wait_recv()
                inflight.append(ag[p])
                if lvl > 0:
                    nxt[p] = start_ag(lvl - 1, p)
            ag = nxt

        for rdma in inflight:
            rdma.wait_send()

    n_sems = 6 * n_parts
    return pl.pallas_call(
        body,
        out_shape=jax.ShapeDtypeStruct((m, n), x.dtype),
        in_specs=[pl.BlockSpec(memory_space=pltpu.VMEM)],
        out_specs=pl.BlockSpec(memory_space=pltpu.VMEM),
        scratch_shapes=[
            pltpu.VMEM((comm_rows, n), x.dtype),
            pltpu.SemaphoreType.DMA((n_sems,)),
            pltpu.SemaphoreType.DMA((n_sems,)),
        ],
        compiler_params=pltpu.CompilerParams(collective_id=0),
    )(x)


# device time: 26111 ns/iter; 2.7142x vs baseline; 1.1536x over previous
import jax
import jax.numpy as jnp
from jax import lax
from jax.experimental import pallas as pl
from jax.experimental.pallas import tpu as pltpu

N_DEV = 8
M, N = 1024, 512

PARTS = (
    (0, 384, ("x", "y", "z")),
    (384, 320, ("y", "z", "x")),
    (704, 320, ("z", "x", "y")),
)
SPLITS = 2
CW = N // SPLITS


def kernel(x):
    m, n = x.shape
    assert (m, n) == (M, N)
    n_parts = len(PARTS)

    comm_offs = []
    off = 0
    for _, size, _ in PARTS:
        offs = []
        for frac in (2, 4, 8):
            offs.append(off)
            off += size // frac
        comm_offs.append(offs)
    comm_rows = off

    def body(x_ref, out_ref, comm_ref, send_sems, recv_sems):
        my = lax.axis_index("i")
        bz = my // 4
        q = lax.rem(my, 4)
        by = q // 2
        bx = jnp.bitwise_xor(by, lax.rem(q, 2))

        bits = {"x": bx, "y": by, "z": bz}
        partners = {
            "x": bz * 4 + by * 2 + jnp.bitwise_xor(1 - bx, by),
            "y": bz * 4 + (1 - by) * 2 + jnp.bitwise_xor(bx, 1 - by),
            "z": (1 - bz) * 4 + by * 2 + jnp.bitwise_xor(bx, by),
        }

        barrier_sem = pltpu.get_barrier_semaphore()
        for ax in ("x", "y", "z"):
            pl.semaphore_signal(
                barrier_sem, inc=1,
                device_id=(partners[ax],), device_id_type=pl.DeviceIdType.MESH,
            )
        pl.semaphore_wait(barrier_sem, 3)

        keep_offs = [[None] * 3 for _ in range(n_parts)]
        send_offs = [[None] * 3 for _ in range(n_parts)]
        for p, (base, size, order) in enumerate(PARTS):
            cur = base
            for lvl, ax in enumerate(order):
                half = size >> (lvl + 1)
                keep_offs[p][lvl] = cur + bits[ax] * half
                send_offs[p][lvl] = cur + (1 - bits[ax]) * half
                cur = keep_offs[p][lvl]

        def sem_idx(stage, p, h):
            return (stage * n_parts + p) * SPLITS + h

        def start_rs(lvl, p, h):
            _, size, order = PARTS[p]
            sz = size >> (lvl + 1)
            cols = pl.ds(h * CW, CW)
            src = x_ref if lvl == 0 else out_ref
            rdma = pltpu.make_async_remote_copy(
                src_ref=src.at[pl.ds(send_offs[p][lvl], sz), cols],
                dst_ref=comm_ref.at[pl.ds(comm_offs[p][lvl], sz), cols],
                send_sem=send_sems.at[sem_idx(lvl, p, h)],
                recv_sem=recv_sems.at[sem_idx(lvl, p, h)],
                device_id=(partners[order[lvl]],),
                device_id_type=pl.DeviceIdType.MESH,
            )
            rdma.start()
            return rdma

        def start_ag(lvl, p, h):
            _, size, order = PARTS[p]
            sz = size >> (lvl + 1)
            cols = pl.ds(h * CW, CW)
            rdma = pltpu.make_async_remote_copy(
                src_ref=out_ref.at[pl.ds(keep_offs[p][lvl], sz), cols],
                dst_ref=out_ref.at[pl.ds(keep_offs[p][lvl], sz), cols],
                send_sem=send_sems.at[sem_idx(5 - lvl, p, h)],
                recv_sem=recv_sems.at[sem_idx(5 - lvl, p, h)],
                device_id=(partners[order[lvl]],),
                device_id_type=pl.DeviceIdType.MESH,
            )
            rdma.start()
            return rdma

        chains = [(p, h) for h in range(SPLITS) for p in (1, 2, 0)]
        inflight = []

        cur = {ph: start_rs(0, *ph) for ph in chains}
        for lvl in range(3):
            nxt = {}
            for p, h in chains:
                _, size, _ = PARTS[p]
                sz = size >> (lvl + 1)
                cols = pl.ds(h * CW, CW)
                cur[(p, h)].wait_recv()
                inflight.append(cur[(p, h)])
                if lvl == 0:
                    out_ref[pl.ds(keep_offs[p][0], sz), cols] = (
                        x_ref[pl.ds(keep_offs[p][0], sz), cols]
                        + comm_ref[pl.ds(comm_offs[p][0], sz), cols]
                    )
                else:
                    out_ref[pl.ds(keep_offs[p][lvl], sz), cols] += comm_ref[
                        pl.ds(comm_offs[p][lvl], sz), cols
                    ]
                nxt[(p, h)] = (
                    start_rs(lvl + 1, p, h) if lvl < 2 else start_ag(2, p, h)
                )
            cur = nxt

        for lvl in (2, 1, 0):
            nxt = {}
            for p, h in chains:
                cur[(p, h)].wait_recv()
                inflight.append(cur[(p, h)])
                if lvl > 0:
                    nxt[(p, h)] = start_ag(lvl - 1, p, h)
            cur = nxt

        for rdma in inflight:
            rdma.wait_send()

    n_sems = 6 * n_parts * SPLITS
    return pl.pallas_call(
        body,
        out_shape=jax.ShapeDtypeStruct((m, n), x.dtype),
        in_specs=[pl.BlockSpec(memory_space=pltpu.VMEM)],
        out_specs=pl.BlockSpec(memory_space=pltpu.VMEM),
        scratch_shapes=[
            pltpu.VMEM((comm_rows, n), x.dtype),
            pltpu.SemaphoreType.DMA((n_sems,)),
            pltpu.SemaphoreType.DMA((n_sems,)),
        ],
        compiler_params=pltpu.CompilerParams(collective_id=0),
    )(x)
